# baseline (speedup 1.0000x reference)
import functools

import jax
import jax.numpy as jnp
from jax import lax
from jax.experimental import pallas as pl
from jax.experimental.pallas import tpu as pltpu
from jax.experimental.pallas import tpu_sc as plsc

N_NODES = 5000
EMB = 64
N_EDGES = 1_000_000
EPAD = 1_048_576
NC, NS, LANES = 2, 16, 16
NW = NC * NS
PER_TILE = EPAD // NW
CHUNK = 1024
NCHUNKS = PER_TILE // CHUNK
RING = 4
FAST_CHUNKS = 18
SLOW_CHUNKS = 14
SLOW_CORE = 1
MAX_STEPS = 20


def _node_tab_body(src_ref, dst_ref, wa_ref, wc_ref, node_ref):
    dims = (((1,), (1,)), ((), ()))
    a_row = lax.dot_general(wa_ref[...], src_ref[...], dims,
                            preferred_element_type=jnp.float32)
    c_row = lax.dot_general(wc_ref[...], dst_ref[...], dims,
                            preferred_element_type=jnp.float32)
    rs = jnp.max(src_ref[...], axis=1)[None, :]
    rd = jnp.max(dst_ref[...], axis=1)[None, :]
    node_ref[...] = jnp.concatenate([a_row, c_row, rs, rd], axis=0)


def _maxd_body(dist_ref, maxd_ref):
    g = pl.program_id(0)

    @pl.when(g == 0)
    def _():
        maxd_ref[...] = jnp.full((1, 1), -jnp.inf, jnp.float32)

    blk_max = jnp.max(dist_ref[...]).reshape(1, 1)
    maxd_ref[...] = jnp.maximum(maxd_ref[...], blk_max)


def _sc_s_body(ij_hbm, node_hbm, s_hbm, maxes_hbm,
               a_v, c_v, rs_v, rd_v, ij0, ij1, sv0, sv1, mv,
               sem_ij, sem_s):
    ijv, sv = (ij0, ij1), (sv0, sv1)
    wid = lax.axis_index("s") * NC + lax.axis_index("c")
    base = wid * PER_TILE

    pltpu.sync_copy(node_hbm.at[0], a_v)
    pltpu.sync_copy(node_hbm.at[1], c_v)
    pltpu.sync_copy(node_hbm.at[2], rs_v)
    pltpu.sync_copy(node_hbm.at[3], rd_v)

    iota2 = lax.iota(jnp.int32, LANES) * 2

    def ij_copy(t, b):
        off = 2 * (base + t * CHUNK)
        return pltpu.make_async_copy(ij_hbm.at[pl.ds(off, 2 * CHUNK)],
                                     ijv[b], sem_ij.at[b])

    def store_copy(t, b):
        off = base + t * CHUNK
        return pltpu.make_async_copy(sv[b], s_hbm.at[pl.ds(off, CHUNK)],
                                     sem_s.at[b])

    ij_copy(0, 0).start()

    def step(t, b, m):
        @pl.when(t + 1 < NCHUNKS)
        def _():
            ij_copy(t + 1, 1 - b).start()

        ij_copy(t, b).wait()

        @pl.when(t >= 2)
        def _():
            store_copy(t - 2, b).wait()

        for g in range(CHUNK // LANES):
            idx_i = iota2 + (2 * g * LANES)
            idx_j = idx_i + 1
            ii = plsc.load_gather(ijv[b], [idx_i])
            jj = plsc.load_gather(ijv[b], [idx_j])
            sv[b][pl.ds(g * LANES, LANES)] = (plsc.load_gather(a_v, [ii]) +
                                              plsc.load_gather(c_v, [jj]))
            m = jnp.maximum(m, plsc.load_gather(rs_v, [ii]))
            m = jnp.maximum(m, plsc.load_gather(rd_v, [jj]))

        store_copy(t, b).start()
        return m

    def outer(p, m):
        m = step(2 * p, 0, m)
        m = step(2 * p + 1, 1, m)
        return m

    m = lax.fori_loop(0, NCHUNKS // 2, outer,
                      jnp.full((LANES,), -jnp.inf, jnp.float32))

    store_copy(NCHUNKS - 2, 0).wait()
    store_copy(NCHUNKS - 1, 1).wait()

    mv[...] = m
    pltpu.sync_copy(mv, maxes_hbm.at[wid])


def _sc_d_body(ij_hbm, distm_hbm, d_hbm,
               ij0, ij1, ij2, ij3, fv0, fv1, fv2, fv3, dv0, dv1, dv2, dv3,
               sem_ij, sem_g, sem_d):
    ijv = (ij0, ij1, ij2, ij3)
    fv, dv = (fv0, fv1, fv2, fv3), (dv0, dv1, dv2, dv3)
    wid = lax.axis_index("s") * NC + lax.axis_index("c")
    base = wid * PER_TILE

    iota2 = lax.iota(jnp.int32, LANES) * 2

    def ij_copy(t, b):
        off = 2 * (base + t * CHUNK)
        return pltpu.make_async_copy(ij_hbm.at[pl.ds(off, 2 * CHUNK)],
                                     ijv[b], sem_ij.at[b])

    def gather_copy(b):
        return pltpu.make_async_copy(distm_hbm.at[fv[b]], dv[b],
                                     sem_g.at[b])

    def store_copy(t, b):
        off = base + t * CHUNK
        return pltpu.make_async_copy(dv[b], d_hbm.at[pl.ds(off, CHUNK)],
                                     sem_d.at[b])

    for t0 in (0, 1):
        ij_copy(t0, t0).start()

    def step(t, b):
        @pl.when(t + 2 < NCHUNKS)
        def _():
            ij_copy(t + 2, (b + 2) % RING).start()

        ij_copy(t, b).wait()

        @pl.when(t >= RING)
        def _():
            store_copy(t - RING, b).wait()

        for g in range(CHUNK // LANES):
            idx_i = iota2 + (2 * g * LANES)
            ii = plsc.load_gather(ijv[b], [idx_i])
            jj = plsc.load_gather(ijv[b], [idx_i + 1])
            fv[b][pl.ds(g * LANES, LANES)] = ii * N_NODES + jj

        gather_copy(b).start()

        @pl.when(t >= 2)
        def _():
            gather_copy((b - 2) % RING).wait()
            store_copy(t - 2, (b - 2) % RING).start()

    def outer(p, carry):
        for q in range(RING):
            step(RING * p + q, q)
        return carry

    lax.fori_loop(0, NCHUNKS // RING, outer, 0)

    for t in (NCHUNKS - 1, NCHUNKS):
        b = (t - 1) % RING
        gather_copy(b).wait()
        store_copy(t - 1, b).start()
    for t in range(NCHUNKS - RING, NCHUNKS):
        store_copy(t, t % RING).wait()


def _combine_body(s_ref, d_ref, maxes_ref, maxd_ref, wmid_ref, b_ref,
                  out_ref):
    kscale = jnp.max(maxes_ref[...]) * wmid_ref[0, 0] / maxd_ref[0, 0]
    out_ref[...] = s_ref[...] + kscale * d_ref[...] + b_ref[0, 0]


@jax.jit
def kernel(trip_od, src_embedding, dst_embedding, distm, w, b):
    trip = trip_od.astype(jnp.int32)
    npad = EPAD - N_EDGES
    ij = jnp.concatenate([trip.reshape(-1),
                          jnp.broadcast_to(trip[0], (npad, 2)).reshape(-1)])
    wa = w[:EMB].reshape(1, EMB)
    wc = w[EMB + 1:].reshape(1, EMB)
    wmid = w[EMB].reshape(1, 1)
    b2 = b.reshape(1, 1)

    node_tab = pl.pallas_call(
        _node_tab_body,
        in_specs=[
            pl.BlockSpec((N_NODES, EMB), lambda: (0, 0)),
            pl.BlockSpec((N_NODES, EMB), lambda: (0, 0)),
            pl.BlockSpec((1, EMB), lambda: (0, 0)),
            pl.BlockSpec((1, EMB), lambda: (0, 0)),
        ],
        out_specs=pl.BlockSpec((4, N_NODES), lambda: (0, 0)),
        out_shape=jax.ShapeDtypeStruct((4, N_NODES), jnp.float32),
    )(src_embedding, dst_embedding, wa, wc)

    sc_s = functools.partial(
        pl.kernel,
        out_type=(
            jax.ShapeDtypeStruct((EPAD,), jnp.float32),
            jax.ShapeDtypeStruct((NW, LANES), jnp.float32),
        ),
        mesh=plsc.VectorSubcoreMesh(core_axis_name="c", subcore_axis_name="s"),
        compiler_params=pltpu.CompilerParams(needs_layout_passes=False),
        scratch_types=(
            [pltpu.VMEM((N_NODES,), jnp.float32)] * 4 +
            [pltpu.VMEM((2 * CHUNK,), jnp.int32)] * 2 +
            [pltpu.VMEM((CHUNK,), jnp.float32)] * 2 +
            [pltpu.VMEM((LANES,), jnp.float32)] +
            [pltpu.SemaphoreType.DMA((2,))] * 2
        ),
    )(_sc_s_body)
    s_e, tile_maxes = sc_s(ij, node_tab)

    distm_flat = distm.reshape(-1)

    sc_d = functools.partial(
        pl.kernel,
        out_type=jax.ShapeDtypeStruct((EPAD,), jnp.float32),
        mesh=plsc.VectorSubcoreMesh(core_axis_name="c", subcore_axis_name="s"),
        compiler_params=pltpu.CompilerParams(needs_layout_passes=False),
        scratch_types=(
            [pltpu.VMEM((2 * CHUNK,), jnp.int32)] * 4 +
            [pltpu.VMEM((CHUNK,), jnp.int32)] * 4 +
            [pltpu.VMEM((CHUNK,), jnp.float32)] * 4 +
            [pltpu.SemaphoreType.DMA((RING,))] * 3
        ),
    )(_sc_d_body)
    d_e = sc_d(ij, distm_flat)

    rows_blk = 200
    maxd = pl.pallas_call(
        _maxd_body,
        grid=(N_NODES // rows_blk,),
        in_specs=[pl.BlockSpec((rows_blk, N_NODES), lambda g: (g, 0))],
        out_specs=pl.BlockSpec((1, 1), lambda g: (0, 0)),
        out_shape=jax.ShapeDtypeStruct((1, 1), jnp.float32),
    )(distm)

    blk = EPAD // 8
    out = pl.pallas_call(
        _combine_body,
        grid=(8,),
        in_specs=[
            pl.BlockSpec((blk,), lambda g: (g,)),
            pl.BlockSpec((blk,), lambda g: (g,)),
            pl.BlockSpec((NW, LANES), lambda g: (0, 0)),
            pl.BlockSpec((1, 1), lambda g: (0, 0)),
            pl.BlockSpec((1, 1), lambda g: (0, 0)),
            pl.BlockSpec((1, 1), lambda g: (0, 0)),
        ],
        out_specs=pl.BlockSpec((blk,), lambda g: (g,)),
        out_shape=jax.ShapeDtypeStruct((EPAD,), jnp.float32),
    )(s_e, d_e, tile_maxes, maxd, wmid, b2)

    return out[:N_EDGES]


# speedup vs baseline: 7.1510x; 7.1510x over previous
import functools

import jax
import jax.numpy as jnp
from jax import lax
from jax.experimental import pallas as pl
from jax.experimental.pallas import tpu as pltpu
from jax.experimental.pallas import tpu_sc as plsc

N_NODES = 5000
EMB = 64
N_EDGES = 1_000_000
EPAD = 1_048_576
NC, NS, LANES = 2, 16, 16
NW = NC * NS
PER_TILE = EPAD // NW
CHUNK = 1024
NCHUNKS = PER_TILE // CHUNK
RING = 4
FAST_CHUNKS = 19
SLOW_CHUNKS = 13
SLOW_CORE = 1


def _node_tab_body(src_ref, dst_ref, wa_ref, wc_ref, node_ref):
    dims = (((1,), (1,)), ((), ()))
    a_row = lax.dot_general(wa_ref[...], src_ref[...], dims,
                            preferred_element_type=jnp.float32)
    c_row = lax.dot_general(wc_ref[...], dst_ref[...], dims,
                            preferred_element_type=jnp.float32)
    rs = jnp.max(src_ref[...], axis=1)[None, :]
    rd = jnp.max(dst_ref[...], axis=1)[None, :]
    node_ref[...] = jnp.concatenate([a_row, c_row, rs, rd], axis=0)


def _maxd_body(dist_ref, maxd_ref):
    g = pl.program_id(0)

    @pl.when(g == 0)
    def _():
        maxd_ref[...] = jnp.full((1, 1), -jnp.inf, jnp.float32)

    blk_max = jnp.max(dist_ref[...]).reshape(1, 1)
    maxd_ref[...] = jnp.maximum(maxd_ref[...], blk_max)


def _sc_s_body(i_hbm, j_hbm, node_hbm, s_hbm, maxes_hbm,
               a_v, c_v, rs_v, rd_v, iv0, iv1, jv0, jv1, sv0, sv1, mv,
               sem_i, sem_j, sem_s):
    iv, jv, sv = (iv0, iv1), (jv0, jv1), (sv0, sv1)
    wid = lax.axis_index("s") * NC + lax.axis_index("c")
    base = wid * PER_TILE

    pltpu.sync_copy(node_hbm.at[0], a_v)
    pltpu.sync_copy(node_hbm.at[1], c_v)
    pltpu.sync_copy(node_hbm.at[2], rs_v)
    pltpu.sync_copy(node_hbm.at[3], rd_v)

    def idx_copies(t, b):
        off = base + t * CHUNK
        return (pltpu.make_async_copy(i_hbm.at[pl.ds(off, CHUNK)], iv[b],
                                      sem_i.at[b]),
                pltpu.make_async_copy(j_hbm.at[pl.ds(off, CHUNK)], jv[b],
                                      sem_j.at[b]))

    def store_copy(t, b):
        off = base + t * CHUNK
        return pltpu.make_async_copy(sv[b], s_hbm.at[pl.ds(off, CHUNK)],
                                     sem_s.at[b])

    for cp in idx_copies(0, 0):
        cp.start()

    def step(t, b, m):
        @pl.when(t + 1 < NCHUNKS)
        def _():
            for cp in idx_copies(t + 1, 1 - b):
                cp.start()

        for cp in idx_copies(t, b):
            cp.wait()

        @pl.when(t >= 2)
        def _():
            store_copy(t - 2, b).wait()

        for g in range(CHUNK // LANES):
            o = g * LANES
            ii = iv[b][pl.ds(o, LANES)]
            jj = jv[b][pl.ds(o, LANES)]
            sv[b][pl.ds(o, LANES)] = (plsc.load_gather(a_v, [ii]) +
                                      plsc.load_gather(c_v, [jj]))
            m = jnp.maximum(m, plsc.load_gather(rs_v, [ii]))
            m = jnp.maximum(m, plsc.load_gather(rd_v, [jj]))

        store_copy(t, b).start()
        return m

    def outer(p, m):
        m = step(2 * p, 0, m)
        m = step(2 * p + 1, 1, m)
        return m

    m = lax.fori_loop(0, NCHUNKS // 2, outer,
                      jnp.full((LANES,), -jnp.inf, jnp.float32))

    store_copy(NCHUNKS - 2, 0).wait()
    store_copy(NCHUNKS - 1, 1).wait()

    mv[...] = m
    pltpu.sync_copy(mv, maxes_hbm.at[wid])


def _sc_d_body(i_hbm, j_hbm, distm_hbm, d_hbm,
               iv0, iv1, iv2, iv3, jv0, jv1, jv2, jv3,
               fv0, fv1, fv2, fv3, dv0, dv1, dv2, dv3,
               sem_i, sem_j, sem_g, sem_d):
    iv, jv = (iv0, iv1, iv2, iv3), (jv0, jv1, jv2, jv3)
    fv, dv = (fv0, fv1, fv2, fv3), (dv0, dv1, dv2, dv3)
    cid = lax.axis_index("c")
    sid = lax.axis_index("s")
    region = sid * (FAST_CHUNKS + SLOW_CHUNKS) * CHUNK

    def make_pipeline(n_chunks, base):
        def idx_copies(t, b):
            off = base + t * CHUNK
            return (pltpu.make_async_copy(i_hbm.at[pl.ds(off, CHUNK)],
                                          iv[b], sem_i.at[b]),
                    pltpu.make_async_copy(j_hbm.at[pl.ds(off, CHUNK)],
                                          jv[b], sem_j.at[b]))

        def gather_copy(b):
            return pltpu.make_async_copy(distm_hbm.at[fv[b]], dv[b],
                                         sem_g.at[b])

        def store_copy(t, b):
            off = base + t * CHUNK
            return pltpu.make_async_copy(dv[b], d_hbm.at[pl.ds(off, CHUNK)],
                                         sem_d.at[b])

        def compute_chunk(b):
            def body(g, carry):
                o = g * LANES
                ii = iv[b][pl.ds(o, LANES)]
                jj = jv[b][pl.ds(o, LANES)]
                fv[b][pl.ds(o, LANES)] = ii * N_NODES + jj
                return carry

            lax.fori_loop(0, CHUNK // LANES, body, 0)

        def step(t, b):
            if t + 2 < n_chunks:
                for cp in idx_copies(t + 2, (b + 2) % RING):
                    cp.start()

            for cp in idx_copies(t, b):
                cp.wait()

            if t >= RING:
                store_copy(t - RING, b).wait()

            compute_chunk(b)
            gather_copy(b).start()

            if t >= 2:
                gather_copy((b - 2) % RING).wait()
                store_copy(t - 2, (b - 2) % RING).start()

        def run():
            for t0 in (0, 1):
                for cp in idx_copies(t0, t0):
                    cp.start()
            for t in range(n_chunks):
                step(t, t % RING)
            for t in (n_chunks - 1, n_chunks):
                b = (t - 1) % RING
                gather_copy(b).wait()
                store_copy(t - 1, b).start()
            for t in range(max(n_chunks - RING, 0), n_chunks):
                store_copy(t, t % RING).wait()

        return run

    lax.cond(cid == SLOW_CORE,
             make_pipeline(SLOW_CHUNKS, region + FAST_CHUNKS * CHUNK),
             make_pipeline(FAST_CHUNKS, region))


def _combine_body(s_ref, d_ref, maxes_ref, maxd_ref, wmid_ref, b_ref,
                  out_ref):
    kscale = jnp.max(maxes_ref[...]) * wmid_ref[0, 0] / maxd_ref[0, 0]
    out_ref[...] = s_ref[...] + kscale * d_ref[...] + b_ref[0, 0]


@jax.jit
def kernel(trip_od, src_embedding, dst_embedding, distm, w, b):
    trip = trip_od.astype(jnp.int32)
    npad = EPAD - N_EDGES
    iv = jnp.concatenate([trip[:, 0], jnp.broadcast_to(trip[0, 0], (npad,))])
    jv = jnp.concatenate([trip[:, 1], jnp.broadcast_to(trip[0, 1], (npad,))])
    wa = w[:EMB].reshape(1, EMB)
    wc = w[EMB + 1:].reshape(1, EMB)
    wmid = w[EMB].reshape(1, 1)
    b2 = b.reshape(1, 1)

    node_tab = pl.pallas_call(
        _node_tab_body,
        in_specs=[
            pl.BlockSpec((N_NODES, EMB), lambda: (0, 0)),
            pl.BlockSpec((N_NODES, EMB), lambda: (0, 0)),
            pl.BlockSpec((1, EMB), lambda: (0, 0)),
            pl.BlockSpec((1, EMB), lambda: (0, 0)),
        ],
        out_specs=pl.BlockSpec((4, N_NODES), lambda: (0, 0)),
        out_shape=jax.ShapeDtypeStruct((4, N_NODES), jnp.float32),
    )(src_embedding, dst_embedding, wa, wc)

    sc_s = functools.partial(
        pl.kernel,
        out_type=(
            jax.ShapeDtypeStruct((EPAD,), jnp.float32),
            jax.ShapeDtypeStruct((NW, LANES), jnp.float32),
        ),
        mesh=plsc.VectorSubcoreMesh(core_axis_name="c", subcore_axis_name="s"),
        compiler_params=pltpu.CompilerParams(needs_layout_passes=False),
        scratch_types=(
            [pltpu.VMEM((N_NODES,), jnp.float32)] * 4 +
            [pltpu.VMEM((CHUNK,), jnp.int32)] * 4 +
            [pltpu.VMEM((CHUNK,), jnp.float32)] * 2 +
            [pltpu.VMEM((LANES,), jnp.float32)] +
            [pltpu.SemaphoreType.DMA((2,))] * 3
        ),
    )(_sc_s_body)
    s_e, tile_maxes = sc_s(iv, jv, node_tab)

    distm_flat = distm.reshape(-1)

    sc_d = functools.partial(
        pl.kernel,
        out_type=jax.ShapeDtypeStruct((EPAD,), jnp.float32),
        mesh=plsc.VectorSubcoreMesh(core_axis_name="c", subcore_axis_name="s"),
        compiler_params=pltpu.CompilerParams(needs_layout_passes=False),
        scratch_types=(
            [pltpu.VMEM((CHUNK,), jnp.int32)] * 12 +
            [pltpu.VMEM((CHUNK,), jnp.float32)] * 4 +
            [pltpu.SemaphoreType.DMA((RING,))] * 4
        ),
    )(_sc_d_body)
    d_e = sc_d(iv, jv, distm_flat)

    rows_blk = 200
    maxd = pl.pallas_call(
        _maxd_body,
        grid=(N_NODES // rows_blk,),
        in_specs=[pl.BlockSpec((rows_blk, N_NODES), lambda g: (g, 0))],
        out_specs=pl.BlockSpec((1, 1), lambda g: (0, 0)),
        out_shape=jax.ShapeDtypeStruct((1, 1), jnp.float32),
    )(distm)

    blk = EPAD // 8
    out = pl.pallas_call(
        _combine_body,
        grid=(8,),
        in_specs=[
            pl.BlockSpec((blk,), lambda g: (g,)),
            pl.BlockSpec((blk,), lambda g: (g,)),
            pl.BlockSpec((NW, LANES), lambda g: (0, 0)),
            pl.BlockSpec((1, 1), lambda g: (0, 0)),
            pl.BlockSpec((1, 1), lambda g: (0, 0)),
            pl.BlockSpec((1, 1), lambda g: (0, 0)),
        ],
        out_specs=pl.BlockSpec((blk,), lambda g: (g,)),
        out_shape=jax.ShapeDtypeStruct((EPAD,), jnp.float32),
    )(s_e, d_e, tile_maxes, maxd, wmid, b2)

    return out[:N_EDGES]
